# baseline (speedup 1.0000x reference)
import functools

import jax
import jax.numpy as jnp
import numpy as np
from jax import lax
from jax.experimental import pallas as pl
from jax.experimental.pallas import tpu as pltpu
from jax.experimental.pallas import tpu_sc as plsc

VOCAB = 1000000
D_MODEL = 64
SEQ = 200
BATCH = 1024

NUM_CORES = 2
NUM_SUBCORES = 16
NW = NUM_CORES * NUM_SUBCORES
B = BATCH * SEQ
B_PER_W = B // NW
CHUNK = 800
NCHUNK = B_PER_W // CHUNK
SEQ_PER_CHUNK = CHUNK // SEQ
D_VECS = D_MODEL // 16


def _positional_encoding(length, depth):
    depth_h = depth / 2
    positions = np.arange(length)[:, np.newaxis]
    depths = np.arange(depth_h)[np.newaxis, :] / depth_h
    angle_rates = 1 / 10000 ** depths
    angle_rads = positions * angle_rates
    return np.concatenate(
        [np.sin(angle_rads), np.cos(angle_rads)], axis=-1
    ).astype(np.float32)


_POS_NP = _positional_encoding(SEQ, D_MODEL)


def _sc_kernel(table_hbm, idx_hbm, pos_hbm, out_hbm, idx_v, rows_v, pos_v, sem):
    wid = lax.axis_index("s") * NUM_CORES + lax.axis_index("c")
    base = wid * B_PER_W

    pltpu.sync_copy(pos_hbm, pos_v)

    scale = jnp.float32(8.0)

    def chunk_body(c, _):
        cbase = base + c * CHUNK
        pltpu.sync_copy(idx_hbm.at[pl.ds(cbase, CHUNK)], idx_v)
        pltpu.async_copy(table_hbm.at[idx_v], rows_v, sem).wait()

        def pos_body(p, _):
            for s in range(SEQ_PER_CHUNK):
                r = s * SEQ + p
                for d in range(D_VECS):
                    sl = pl.ds(d * 16, 16)
                    rows_v[r, sl] = rows_v[r, sl] * scale + pos_v[p, sl]
            return 0

        lax.fori_loop(0, SEQ, pos_body, 0)
        pltpu.sync_copy(rows_v, out_hbm.at[pl.ds(cbase, CHUNK)])
        return 0

    lax.fori_loop(0, NCHUNK, chunk_body, 0)


@jax.jit
def _run(x, table):
    idx = x.reshape(-1)
    mesh = plsc.VectorSubcoreMesh(core_axis_name="c", subcore_axis_name="s")
    k = functools.partial(
        pl.kernel,
        out_type=jax.ShapeDtypeStruct((B, D_MODEL), jnp.float32),
        mesh=mesh,
        scratch_types=[
            pltpu.VMEM((CHUNK,), jnp.int32),
            pltpu.VMEM((CHUNK, D_MODEL), jnp.float32),
            pltpu.VMEM((SEQ, D_MODEL), jnp.float32),
            pltpu.SemaphoreType.DMA,
        ],
        compiler_params=pltpu.CompilerParams(use_tc_tiling_on_sc=False),
    )(_sc_kernel)
    out = k(table, idx, jnp.asarray(_POS_NP))
    return out.reshape(BATCH, SEQ, D_MODEL)


def kernel(x, table):
    return _run(x, table)
